# baseline (speedup 1.0000x reference)
import functools

import jax
import jax.numpy as jnp
import numpy as np
from jax import lax
from jax.experimental import pallas as pl
from jax.experimental.pallas import tpu as pltpu
from jax.experimental.pallas import tpu_sc as plsc

EMBED = 1024
HEADS = 16
SEQ = 2048
DH = EMBED // HEADS
FF = 4 * EMBED
NUM_CLASSES = 50
TOKEN_SELF_ATTN_VALUE = -5e4


def _pos_encoding_np(seq_len, dim):
    pos = np.arange(seq_len)[:, None].astype(np.float32)
    i = np.arange(dim // 2)[None, :].astype(np.float32)
    angle = pos / np.power(10000.0, (2.0 * i) / dim)
    pe = np.zeros((seq_len, dim), dtype=np.float32)
    pe[:, 0::2] = np.sin(angle)
    pe[:, 1::2] = np.cos(angle)
    return pe


def _sc_gather(table, ids):
    info = plsc.get_sparse_core_info()
    nc, ns = info.num_cores, info.num_subcores
    nw = nc * ns
    b = ids.shape[0]
    d = table.shape[1]
    bpw = b // nw
    mesh = plsc.VectorSubcoreMesh(core_axis_name="c", subcore_axis_name="s")

    @functools.partial(
        pl.kernel, mesh=mesh,
        out_type=jax.ShapeDtypeStruct((b, d), jnp.float32),
        scratch_types=[
            pltpu.VMEM((bpw,), jnp.int32),
            pltpu.VMEM((bpw, d), jnp.float32),
            pltpu.SemaphoreType.DMA,
        ],
    )
    def k(table_hbm, idx_hbm, out_hbm, idx_v, rows_v, sem):
        wid = lax.axis_index("s") * nc + lax.axis_index("c")
        base = wid * bpw
        pltpu.sync_copy(idx_hbm.at[pl.ds(base, bpw)], idx_v)
        pltpu.async_copy(table_hbm.at[idx_v], rows_v, sem).wait()
        pltpu.sync_copy(rows_v, out_hbm.at[pl.ds(base, bpw)])

    return k(table, ids)


def _add_body(a_ref, b_ref, o_ref):
    o_ref[...] = a_ref[...] + b_ref[...]


def _add(a, b):
    return pl.pallas_call(
        _add_body,
        out_shape=jax.ShapeDtypeStruct(a.shape, jnp.float32),
    )(a, b)


def _layer_norm(x, g, b):
    mu = jnp.mean(x, axis=-1, keepdims=True)
    var = jnp.mean((x - mu) ** 2, axis=-1, keepdims=True)
    return (x - mu) / jnp.sqrt(var + 1e-5) * g + b


def _proj_body(x_ref, g_ref, bb_ref, qkw_ref, qkb_ref, vw_ref, vb_ref,
               qk_ref, v_ref):
    xn = _layer_norm(x_ref[...], g_ref[...], bb_ref[...]).astype(jnp.bfloat16)
    qk_ref[...] = (
        jnp.dot(xn, qkw_ref[...], preferred_element_type=jnp.float32)
        + qkb_ref[...])
    v_ref[...] = (
        jnp.dot(xn, vw_ref[...], preferred_element_type=jnp.float32)
        + vb_ref[...]).astype(jnp.bfloat16)


def _proj(x, g, b, qkw, qkb, vw, vb):
    nblk = 4
    rows = SEQ // nblk
    return pl.pallas_call(
        _proj_body,
        grid=(nblk,),
        in_specs=[
            pl.BlockSpec((rows, EMBED), lambda i: (i, 0)),
            pl.BlockSpec((1, EMBED), lambda i: (0, 0)),
            pl.BlockSpec((1, EMBED), lambda i: (0, 0)),
            pl.BlockSpec((EMBED, EMBED), lambda i: (0, 0)),
            pl.BlockSpec((1, EMBED), lambda i: (0, 0)),
            pl.BlockSpec((EMBED, EMBED), lambda i: (0, 0)),
            pl.BlockSpec((1, EMBED), lambda i: (0, 0)),
        ],
        out_specs=[
            pl.BlockSpec((rows, EMBED), lambda i: (i, 0)),
            pl.BlockSpec((rows, EMBED), lambda i: (i, 0)),
        ],
        out_shape=[
            jax.ShapeDtypeStruct((SEQ, EMBED), jnp.float32),
            jax.ShapeDtypeStruct((SEQ, EMBED), jnp.bfloat16),
        ],
    )(x, g, b, qkw, qkb, vw, vb)


_QROWS = 512
_HP = 2


def _attn_body(qk_q_ref, qk_k_ref, v_ref, kbias_ref, o_ref):
    j = pl.program_id(1)
    kbias = kbias_ref[...]
    r = lax.broadcasted_iota(jnp.int32, (_QROWS, SEQ), 0) + j * _QROWS
    c = lax.broadcasted_iota(jnp.int32, (_QROWS, SEQ), 1)
    diag = r == c
    for t in range(_HP):
        sl = slice(t * DH, (t + 1) * DH)
        q = qk_q_ref[:, sl].astype(jnp.bfloat16)
        kk = qk_k_ref[:, sl]
        nrm = jnp.sqrt(jnp.sum(kk * kk, axis=-1, keepdims=True))
        kn = (kk / jnp.maximum(nrm, 1e-8)).astype(jnp.bfloat16)
        dots = lax.dot_general(
            q, kn, (((1,), (1,)), ((), ())),
            preferred_element_type=jnp.float32) * (DH ** -0.5) + kbias
        dots = jnp.where(diag, TOKEN_SELF_ATTN_VALUE, dots)
        m = jnp.max(dots, axis=-1, keepdims=True)
        p = jnp.exp(dots - m)
        attn = (p / jnp.sum(p, axis=-1, keepdims=True)).astype(jnp.bfloat16)
        o_ref[:, sl] = jnp.dot(attn, v_ref[:, sl],
                               preferred_element_type=jnp.float32
                               ).astype(jnp.bfloat16)


def _attention(qk, v, kbias):
    return pl.pallas_call(
        _attn_body,
        grid=(HEADS // _HP, SEQ // _QROWS),
        in_specs=[
            pl.BlockSpec((_QROWS, _HP * DH), lambda hp, j: (j, hp)),
            pl.BlockSpec((SEQ, _HP * DH), lambda hp, j: (0, hp)),
            pl.BlockSpec((SEQ, _HP * DH), lambda hp, j: (0, hp)),
            pl.BlockSpec((1, SEQ), lambda hp, j: (0, 0)),
        ],
        out_specs=pl.BlockSpec((_QROWS, _HP * DH), lambda hp, j: (j, hp)),
        out_shape=jax.ShapeDtypeStruct((SEQ, EMBED), jnp.bfloat16),
    )(qk, qk, v, kbias)


def _ffn_body(a_ref, x1_ref, x2_ref, ow_ref, ob_ref, g_ref, bb_ref,
              f1w_ref, f1b_ref, f2w_ref, f2b_ref, y1_ref, y2_ref):
    a = (jnp.dot(a_ref[...], ow_ref[...], preferred_element_type=jnp.float32)
         + ob_ref[...])
    y1 = x1_ref[...] + a
    y1_ref[...] = y1
    h = _layer_norm(y1, g_ref[...], bb_ref[...]).astype(jnp.bfloat16)
    f = jax.nn.gelu(
        jnp.dot(h, f1w_ref[...], preferred_element_type=jnp.float32)
        + f1b_ref[...]).astype(jnp.bfloat16)
    f = (jnp.dot(f, f2w_ref[...], preferred_element_type=jnp.float32)
         + f2b_ref[...])
    y2_ref[...] = x2_ref[...] + f


def _ffn(attn_out, x1, x2, ow, ob, g, b, f1w, f1b, f2w, f2b):
    nblk = 8
    rows = SEQ // nblk
    return pl.pallas_call(
        _ffn_body,
        grid=(nblk,),
        in_specs=[
            pl.BlockSpec((rows, EMBED), lambda i: (i, 0)),
            pl.BlockSpec((rows, EMBED), lambda i: (i, 0)),
            pl.BlockSpec((rows, EMBED), lambda i: (i, 0)),
            pl.BlockSpec((EMBED, EMBED), lambda i: (0, 0)),
            pl.BlockSpec((1, EMBED), lambda i: (0, 0)),
            pl.BlockSpec((1, EMBED), lambda i: (0, 0)),
            pl.BlockSpec((1, EMBED), lambda i: (0, 0)),
            pl.BlockSpec((EMBED, FF), lambda i: (0, 0)),
            pl.BlockSpec((1, FF), lambda i: (0, 0)),
            pl.BlockSpec((FF, EMBED), lambda i: (0, 0)),
            pl.BlockSpec((1, EMBED), lambda i: (0, 0)),
        ],
        out_specs=[
            pl.BlockSpec((rows, EMBED), lambda i: (i, 0)),
            pl.BlockSpec((rows, EMBED), lambda i: (i, 0)),
        ],
        out_shape=[
            jax.ShapeDtypeStruct((SEQ, EMBED), jnp.float32),
            jax.ShapeDtypeStruct((SEQ, EMBED), jnp.float32),
        ],
    )(attn_out, x1, x2, ow, ob, g, b, f1w, f1b, f2w, f2b)


def _final_body(x1_ref, x2_ref, keep_ref, pw_ref, pb_ref, cw_ref, cb_ref,
                o_ref):
    keep = keep_ref[...]
    hidden = (x1_ref[...] + x2_ref[...]) * 0.5 * keep
    pooled = jnp.sum(hidden, axis=0, keepdims=True) / jnp.sum(keep)
    pre = jnp.maximum(
        jnp.dot(pooled, pw_ref[...], preferred_element_type=jnp.float32)
        + pb_ref[...], 0.0)
    o_ref[...] = (
        jnp.dot(pre, cw_ref[...], preferred_element_type=jnp.float32)
        + cb_ref[...])


def _final(x1, x2, keep, pw, pb, cw, cb):
    return pl.pallas_call(
        _final_body,
        out_shape=jax.ShapeDtypeStruct((1, NUM_CLASSES), jnp.float32),
    )(x1, x2, keep, pw, pb, cw, cb)


def kernel(src, source_lengths, params):
    ids = src.reshape(-1)
    rows = _sc_gather(params["embed"], ids)
    pe = jnp.asarray(_pos_encoding_np(SEQ, EMBED))
    x = _add(rows, pe)

    padf = (ids == 0).astype(jnp.float32)
    kbias = (padf * -1e9).reshape(1, SEQ)
    keep = (1.0 - padf).reshape(SEQ, 1)

    bf = jnp.bfloat16
    x1 = x
    x2 = x
    for lp in params["layers"]:
        qk, v = _proj(
            x2,
            lp["ln1_g"].reshape(1, EMBED), lp["ln1_b"].reshape(1, EMBED),
            lp["qk_w"].astype(bf), lp["qk_b"].reshape(1, EMBED),
            lp["v_w"].astype(bf), lp["v_b"].reshape(1, EMBED))
        attn_out = _attention(qk, v, kbias)
        x1, x2 = _ffn(
            attn_out, x1, x2,
            lp["o_w"].astype(bf), lp["o_b"].reshape(1, EMBED),
            lp["ln2_g"].reshape(1, EMBED), lp["ln2_b"].reshape(1, EMBED),
            lp["ff1_w"].astype(bf), lp["ff1_b"].reshape(1, FF),
            lp["ff2_w"].astype(bf), lp["ff2_b"].reshape(1, EMBED))

    return _final(
        x1, x2, keep,
        params["pre_w"], params["pre_b"].reshape(1, EMBED),
        params["cls_w"], params["cls_b"].reshape(1, NUM_CLASSES))


# speedup vs baseline: 1.6059x; 1.6059x over previous
import functools

import jax
import jax.numpy as jnp
import numpy as np
from jax import lax
from jax.experimental import pallas as pl
from jax.experimental.pallas import tpu as pltpu
from jax.experimental.pallas import tpu_sc as plsc

EMBED = 1024
HEADS = 16
SEQ = 2048
DH = EMBED // HEADS
FF = 4 * EMBED
NUM_CLASSES = 50
TOKEN_SELF_ATTN_VALUE = -5e4


def _pos_encoding_np(seq_len, dim):
    pos = np.arange(seq_len)[:, None].astype(np.float32)
    i = np.arange(dim // 2)[None, :].astype(np.float32)
    angle = pos / np.power(10000.0, (2.0 * i) / dim)
    pe = np.zeros((seq_len, dim), dtype=np.float32)
    pe[:, 0::2] = np.sin(angle)
    pe[:, 1::2] = np.cos(angle)
    return pe


def _sc_gather(table, ids):
    info = plsc.get_sparse_core_info()
    nc, ns = info.num_cores, info.num_subcores
    nw = nc * ns
    b = ids.shape[0]
    d = table.shape[1]
    bpw = b // nw
    mesh = plsc.VectorSubcoreMesh(core_axis_name="c", subcore_axis_name="s")

    @functools.partial(
        pl.kernel, mesh=mesh,
        out_type=jax.ShapeDtypeStruct((b, d), jnp.float32),
        scratch_types=[
            pltpu.VMEM((bpw,), jnp.int32),
            pltpu.VMEM((bpw, d), jnp.float32),
            pltpu.SemaphoreType.DMA,
        ],
    )
    def k(table_hbm, idx_hbm, out_hbm, idx_v, rows_v, sem):
        wid = lax.axis_index("s") * nc + lax.axis_index("c")
        base = wid * bpw
        pltpu.sync_copy(idx_hbm.at[pl.ds(base, bpw)], idx_v)
        pltpu.async_copy(table_hbm.at[idx_v], rows_v, sem).wait()
        pltpu.sync_copy(rows_v, out_hbm.at[pl.ds(base, bpw)])

    return k(table, ids)


def _layer_norm(x, g, b):
    mu = jnp.mean(x, axis=-1, keepdims=True)
    var = jnp.mean((x - mu) ** 2, axis=-1, keepdims=True)
    return (x - mu) / jnp.sqrt(var + 1e-5) * g + b


def _proj_body(x_ref, pe_ref, g_ref, bb_ref, qkw_ref, qkb_ref, vw_ref, vb_ref,
               hm_ref, keep_ref,
               qs_ref, kn_ref, vaug_ref, pdiag_ref, xpe_ref):
    xpe = x_ref[...] + pe_ref[...]
    xpe_ref[...] = xpe
    xn = _layer_norm(xpe, g_ref[...], bb_ref[...])
    qk = (jnp.dot(xn, qkw_ref[...], preferred_element_type=jnp.float32)
          + qkb_ref[...])
    v = (jnp.dot(xn, vw_ref[...], preferred_element_type=jnp.float32)
         + vb_ref[...])
    keep = keep_ref[...]
    sq = (qk * qk).astype(jnp.bfloat16)
    sumsq = jnp.dot(sq, hm_ref[...], preferred_element_type=jnp.float32)
    nrm = jnp.maximum(jnp.sqrt(sumsq), 1e-8)
    qs_ref[...] = (qk * (DH ** -0.5)).astype(jnp.bfloat16)
    kn_ref[...] = (qk * keep / nrm).astype(jnp.bfloat16)
    pdiag_ref[...] = (jnp.exp(jnp.sqrt(sumsq) * (DH ** -0.5))
                      * keep).astype(jnp.bfloat16)
    vk = (v * keep).astype(jnp.bfloat16)
    krep = jnp.broadcast_to(keep, (keep.shape[0], DH)).astype(jnp.bfloat16)
    pieces = []
    for h in range(HEADS):
        pieces.append(vk[:, h * DH:(h + 1) * DH])
        pieces.append(krep)
    vaug_ref[...] = jnp.concatenate(pieces, axis=1)


def _proj(x, pe, g, b, qkw, qkb, vw, vb, hm, keep):
    nblk = 4
    rows = SEQ // nblk
    pe_spec = (pl.BlockSpec((rows, EMBED), lambda i: (i, 0))
               if pe.shape[0] == SEQ
               else pl.BlockSpec((1, EMBED), lambda i: (0, 0)))
    return pl.pallas_call(
        _proj_body,
        grid=(nblk,),
        in_specs=[
            pl.BlockSpec((rows, EMBED), lambda i: (i, 0)),
            pe_spec,
            pl.BlockSpec((1, EMBED), lambda i: (0, 0)),
            pl.BlockSpec((1, EMBED), lambda i: (0, 0)),
            pl.BlockSpec((EMBED, EMBED), lambda i: (0, 0)),
            pl.BlockSpec((1, EMBED), lambda i: (0, 0)),
            pl.BlockSpec((EMBED, EMBED), lambda i: (0, 0)),
            pl.BlockSpec((1, EMBED), lambda i: (0, 0)),
            pl.BlockSpec((EMBED, EMBED), lambda i: (0, 0)),
            pl.BlockSpec((rows, 1), lambda i: (i, 0)),
        ],
        out_specs=[
            pl.BlockSpec((rows, EMBED), lambda i: (i, 0)),
            pl.BlockSpec((rows, EMBED), lambda i: (i, 0)),
            pl.BlockSpec((rows, 2 * EMBED), lambda i: (i, 0)),
            pl.BlockSpec((rows, EMBED), lambda i: (i, 0)),
            pl.BlockSpec((rows, EMBED), lambda i: (i, 0)),
        ],
        out_shape=[
            jax.ShapeDtypeStruct((SEQ, EMBED), jnp.bfloat16),
            jax.ShapeDtypeStruct((SEQ, EMBED), jnp.bfloat16),
            jax.ShapeDtypeStruct((SEQ, 2 * EMBED), jnp.bfloat16),
            jax.ShapeDtypeStruct((SEQ, EMBED), jnp.bfloat16),
            jax.ShapeDtypeStruct((SEQ, EMBED), jnp.float32),
        ],
    )(x, pe, g, b, qkw, qkb, vw, vb, hm, keep)


_QROWS = 1024
_HP = 2


def _attn_body(qs_ref, kn_ref, vaug_ref, vself_ref, pdiag_ref, o_ref):
    for t in range(_HP):
        sl = slice(t * DH, (t + 1) * DH)
        sl2 = slice(t * 2 * DH, t * 2 * DH + DH)
        q = qs_ref[:, sl]
        kk = kn_ref[:, sl]
        dots = lax.dot_general(
            q, kk, (((1,), (1,)), ((), ())),
            preferred_element_type=jnp.float32)
        p = jnp.exp(dots).astype(jnp.bfloat16)
        raw = jnp.dot(p, vaug_ref[:, t * 2 * DH:(t + 1) * 2 * DH],
                      preferred_element_type=jnp.float32)
        pii = pdiag_ref[:, sl].astype(jnp.float32)
        vs = vself_ref[:, sl2].astype(jnp.float32)
        num = raw[:, :DH] - pii * vs
        den = jnp.maximum(raw[:, DH:] - pii, 1e-30)
        o_ref[:, sl] = (num / den).astype(jnp.bfloat16)


def _attention(qs, kn, vaug, pdiag):
    return pl.pallas_call(
        _attn_body,
        grid=(HEADS // _HP, SEQ // _QROWS),
        in_specs=[
            pl.BlockSpec((_QROWS, _HP * DH), lambda hp, j: (j, hp)),
            pl.BlockSpec((SEQ, _HP * DH), lambda hp, j: (0, hp)),
            pl.BlockSpec((SEQ, 2 * _HP * DH), lambda hp, j: (0, hp)),
            pl.BlockSpec((_QROWS, 2 * _HP * DH), lambda hp, j: (j, hp)),
            pl.BlockSpec((_QROWS, _HP * DH), lambda hp, j: (j, hp)),
        ],
        out_specs=pl.BlockSpec((_QROWS, _HP * DH), lambda hp, j: (j, hp)),
        out_shape=jax.ShapeDtypeStruct((SEQ, EMBED), jnp.bfloat16),
    )(qs, kn, vaug, vaug, pdiag)


def _ffn_body(a_ref, x1_ref, x2_ref, ow_ref, ob_ref, g_ref, bb_ref,
              f1w_ref, f1b_ref, f2w_ref, f2b_ref, y1_ref, y2_ref):
    a = (jnp.dot(a_ref[...], ow_ref[...], preferred_element_type=jnp.float32)
         + ob_ref[...])
    y1 = x1_ref[...] + a
    y1_ref[...] = y1
    h = _layer_norm(y1, g_ref[...], bb_ref[...])
    f = jax.nn.gelu(
        jnp.dot(h, f1w_ref[...], preferred_element_type=jnp.float32)
        + f1b_ref[...])
    f = (jnp.dot(f, f2w_ref[...], preferred_element_type=jnp.float32)
         + f2b_ref[...])
    y2_ref[...] = x2_ref[...] + f


def _ffn(attn_out, x1, x2, ow, ob, g, b, f1w, f1b, f2w, f2b):
    nblk = 8
    rows = SEQ // nblk
    return pl.pallas_call(
        _ffn_body,
        grid=(nblk,),
        in_specs=[
            pl.BlockSpec((rows, EMBED), lambda i: (i, 0)),
            pl.BlockSpec((rows, EMBED), lambda i: (i, 0)),
            pl.BlockSpec((rows, EMBED), lambda i: (i, 0)),
            pl.BlockSpec((EMBED, EMBED), lambda i: (0, 0)),
            pl.BlockSpec((1, EMBED), lambda i: (0, 0)),
            pl.BlockSpec((1, EMBED), lambda i: (0, 0)),
            pl.BlockSpec((1, EMBED), lambda i: (0, 0)),
            pl.BlockSpec((EMBED, FF), lambda i: (0, 0)),
            pl.BlockSpec((1, FF), lambda i: (0, 0)),
            pl.BlockSpec((FF, EMBED), lambda i: (0, 0)),
            pl.BlockSpec((1, EMBED), lambda i: (0, 0)),
        ],
        out_specs=[
            pl.BlockSpec((rows, EMBED), lambda i: (i, 0)),
            pl.BlockSpec((rows, EMBED), lambda i: (i, 0)),
        ],
        out_shape=[
            jax.ShapeDtypeStruct((SEQ, EMBED), jnp.float32),
            jax.ShapeDtypeStruct((SEQ, EMBED), jnp.float32),
        ],
    )(attn_out, x1, x2, ow, ob, g, b, f1w, f1b, f2w, f2b)


def _final_body(x1_ref, x2_ref, keep_ref, pw_ref, pb_ref, cw_ref, cb_ref,
                o_ref):
    keep = keep_ref[...]
    hidden = (x1_ref[...] + x2_ref[...]) * 0.5 * keep
    pooled = jnp.sum(hidden, axis=0, keepdims=True) / jnp.sum(keep)
    pre = jnp.maximum(
        jnp.dot(pooled, pw_ref[...], preferred_element_type=jnp.float32)
        + pb_ref[...], 0.0)
    o_ref[...] = (
        jnp.dot(pre, cw_ref[...], preferred_element_type=jnp.float32)
        + cb_ref[...])


def _final(x1, x2, keep, pw, pb, cw, cb):
    return pl.pallas_call(
        _final_body,
        out_shape=jax.ShapeDtypeStruct((1, NUM_CLASSES), jnp.float32),
    )(x1, x2, keep, pw, pb, cw, cb)


def kernel(src, source_lengths, params):
    ids = src.reshape(-1)
    rows = _sc_gather(params["embed"], ids)
    pe_full = jnp.asarray(_pos_encoding_np(SEQ, EMBED))
    pe_zero = jnp.zeros((1, EMBED), jnp.float32)

    padf = (ids == 0).astype(jnp.float32)
    keep = (1.0 - padf).reshape(SEQ, 1)
    hm = jnp.asarray(
        np.kron(np.eye(HEADS, dtype=np.float32),
                np.ones((DH, DH), dtype=np.float32))).astype(jnp.bfloat16)

    x1 = rows
    x2 = rows
    pe = pe_full
    first = True
    for lp in params["layers"]:
        qs, kn, vaug, pdiag, xpe = _proj(
            x2, pe,
            lp["ln1_g"].reshape(1, EMBED), lp["ln1_b"].reshape(1, EMBED),
            lp["qk_w"], lp["qk_b"].reshape(1, EMBED),
            lp["v_w"], lp["v_b"].reshape(1, EMBED),
            hm, keep)
        attn_out = _attention(qs, kn, vaug, pdiag)
        if first:
            x1 = xpe
            x2 = xpe
            first = False
        x1, x2 = _ffn(
            attn_out, x1, x2,
            lp["o_w"], lp["o_b"].reshape(1, EMBED),
            lp["ln2_g"].reshape(1, EMBED), lp["ln2_b"].reshape(1, EMBED),
            lp["ff1_w"], lp["ff1_b"].reshape(1, FF),
            lp["ff2_w"], lp["ff2_b"].reshape(1, EMBED))
        pe = pe_zero

    return _final(
        x1, x2, keep,
        params["pre_w"], params["pre_b"].reshape(1, EMBED),
        params["cls_w"], params["cls_b"].reshape(1, NUM_CLASSES))


# speedup vs baseline: 1.6854x; 1.0495x over previous
import functools

import jax
import jax.numpy as jnp
import numpy as np
from jax import lax
from jax.experimental import pallas as pl
from jax.experimental.pallas import tpu as pltpu
from jax.experimental.pallas import tpu_sc as plsc

EMBED = 1024
HEADS = 16
SEQ = 2048
DH = EMBED // HEADS
FF = 4 * EMBED
NUM_CLASSES = 50
TOKEN_SELF_ATTN_VALUE = -5e4


def _pos_encoding_np(seq_len, dim):
    pos = np.arange(seq_len)[:, None].astype(np.float32)
    i = np.arange(dim // 2)[None, :].astype(np.float32)
    angle = pos / np.power(10000.0, (2.0 * i) / dim)
    pe = np.zeros((seq_len, dim), dtype=np.float32)
    pe[:, 0::2] = np.sin(angle)
    pe[:, 1::2] = np.cos(angle)
    return pe


def _sc_gather(table, ids):
    info = plsc.get_sparse_core_info()
    nc, ns = info.num_cores, info.num_subcores
    nw = nc * ns
    b = ids.shape[0]
    d = table.shape[1]
    bpw = b // nw
    mesh = plsc.VectorSubcoreMesh(core_axis_name="c", subcore_axis_name="s")

    @functools.partial(
        pl.kernel, mesh=mesh,
        out_type=jax.ShapeDtypeStruct((b, d), jnp.float32),
        scratch_types=[
            pltpu.VMEM((bpw,), jnp.int32),
            pltpu.VMEM((bpw, d), jnp.float32),
            pltpu.SemaphoreType.DMA,
        ],
    )
    def k(table_hbm, idx_hbm, out_hbm, idx_v, rows_v, sem):
        wid = lax.axis_index("s") * nc + lax.axis_index("c")
        base = wid * bpw
        pltpu.sync_copy(idx_hbm.at[pl.ds(base, bpw)], idx_v)
        pltpu.async_copy(table_hbm.at[idx_v], rows_v, sem).wait()
        pltpu.sync_copy(rows_v, out_hbm.at[pl.ds(base, bpw)])

    return k(table, ids)


def _layer_norm(x, g, b):
    mu = jnp.mean(x, axis=-1, keepdims=True)
    var = jnp.mean((x - mu) ** 2, axis=-1, keepdims=True)
    return (x - mu) / jnp.sqrt(var + 1e-5) * g + b


def _proj_body(x_ref, pe_ref, g_ref, bb_ref, qkw_ref, qkb_ref, vw_ref, vb_ref,
               hm_ref, keep_ref,
               qs_ref, kn_ref, vaug_ref, pdiag_ref, xpe_ref):
    xpe = x_ref[...].astype(jnp.float32) + pe_ref[...]
    xpe_ref[...] = xpe.astype(jnp.bfloat16)
    xn = _layer_norm(xpe, g_ref[...], bb_ref[...])
    qk = (jnp.dot(xn, qkw_ref[...], preferred_element_type=jnp.float32)
          + qkb_ref[...])
    v = (jnp.dot(xn, vw_ref[...], preferred_element_type=jnp.float32)
         + vb_ref[...])
    keep = keep_ref[...]
    sq = (qk * qk).astype(jnp.bfloat16)
    sumsq = jnp.dot(sq, hm_ref[...], preferred_element_type=jnp.float32)
    nrm = jnp.maximum(jnp.sqrt(sumsq), 1e-8)
    qs_ref[...] = (qk * (DH ** -0.5 * 1.4426950408889634)).astype(jnp.bfloat16)
    kn_ref[...] = (qk * keep / nrm).astype(jnp.bfloat16)
    pdiag_ref[...] = (jnp.exp2(jnp.sqrt(sumsq)
                                * (DH ** -0.5 * 1.4426950408889634))
                      * keep).astype(jnp.bfloat16)
    vk = (v * keep).astype(jnp.bfloat16)
    krep = jnp.broadcast_to(keep, (keep.shape[0], DH)).astype(jnp.bfloat16)
    pieces = []
    for h in range(HEADS):
        pieces.append(vk[:, h * DH:(h + 1) * DH])
        pieces.append(krep)
    vaug_ref[...] = jnp.concatenate(pieces, axis=1)


def _proj(x, pe, g, b, qkw, qkb, vw, vb, hm, keep):
    nblk = 4
    rows = SEQ // nblk
    pe_spec = (pl.BlockSpec((rows, EMBED), lambda i: (i, 0))
               if pe.shape[0] == SEQ
               else pl.BlockSpec((1, EMBED), lambda i: (0, 0)))
    return pl.pallas_call(
        _proj_body,
        grid=(nblk,),
        in_specs=[
            pl.BlockSpec((rows, EMBED), lambda i: (i, 0)),
            pe_spec,
            pl.BlockSpec((1, EMBED), lambda i: (0, 0)),
            pl.BlockSpec((1, EMBED), lambda i: (0, 0)),
            pl.BlockSpec((EMBED, EMBED), lambda i: (0, 0)),
            pl.BlockSpec((1, EMBED), lambda i: (0, 0)),
            pl.BlockSpec((EMBED, EMBED), lambda i: (0, 0)),
            pl.BlockSpec((1, EMBED), lambda i: (0, 0)),
            pl.BlockSpec((EMBED, EMBED), lambda i: (0, 0)),
            pl.BlockSpec((rows, 1), lambda i: (i, 0)),
        ],
        out_specs=[
            pl.BlockSpec((rows, EMBED), lambda i: (i, 0)),
            pl.BlockSpec((rows, EMBED), lambda i: (i, 0)),
            pl.BlockSpec((rows, 2 * EMBED), lambda i: (i, 0)),
            pl.BlockSpec((rows, EMBED), lambda i: (i, 0)),
            pl.BlockSpec((rows, EMBED), lambda i: (i, 0)),
        ],
        out_shape=[
            jax.ShapeDtypeStruct((SEQ, EMBED), jnp.bfloat16),
            jax.ShapeDtypeStruct((SEQ, EMBED), jnp.bfloat16),
            jax.ShapeDtypeStruct((SEQ, 2 * EMBED), jnp.bfloat16),
            jax.ShapeDtypeStruct((SEQ, EMBED), jnp.bfloat16),
            jax.ShapeDtypeStruct((SEQ, EMBED), jnp.bfloat16),
        ],
    )(x, pe, g, b, qkw, qkb, vw, vb, hm, keep)


_QROWS = 2048
_HP = 4


def _attn_body(qs_ref, kn_ref, vaug_ref, vself_ref, pdiag_ref, o_ref):
    for t in range(_HP):
        sl = slice(t * DH, (t + 1) * DH)
        sl2 = slice(t * 2 * DH, t * 2 * DH + DH)
        q = qs_ref[:, sl]
        kk = kn_ref[:, sl]
        dots = lax.dot_general(
            q, kk, (((1,), (1,)), ((), ())),
            preferred_element_type=jnp.float32)
        p = jnp.exp2(dots).astype(jnp.bfloat16)
        raw = jnp.dot(p, vaug_ref[:, t * 2 * DH:(t + 1) * 2 * DH],
                      preferred_element_type=jnp.float32)
        pii = pdiag_ref[:, sl].astype(jnp.float32)
        vs = vself_ref[:, sl2].astype(jnp.float32)
        num = raw[:, :DH] - pii * vs
        den = jnp.maximum(raw[:, DH:] - pii, 1e-30)
        o_ref[:, sl] = (num / den).astype(jnp.bfloat16)


def _attention(qs, kn, vaug, pdiag):
    return pl.pallas_call(
        _attn_body,
        grid=(HEADS // _HP, SEQ // _QROWS),
        in_specs=[
            pl.BlockSpec((_QROWS, _HP * DH), lambda hp, j: (j, hp)),
            pl.BlockSpec((SEQ, _HP * DH), lambda hp, j: (0, hp)),
            pl.BlockSpec((SEQ, 2 * _HP * DH), lambda hp, j: (0, hp)),
            pl.BlockSpec((_QROWS, 2 * _HP * DH), lambda hp, j: (j, hp)),
            pl.BlockSpec((_QROWS, _HP * DH), lambda hp, j: (j, hp)),
        ],
        out_specs=pl.BlockSpec((_QROWS, _HP * DH), lambda hp, j: (j, hp)),
        out_shape=jax.ShapeDtypeStruct((SEQ, EMBED), jnp.bfloat16),
    )(qs, kn, vaug, vaug, pdiag)


def _ffn_body(a_ref, x1_ref, x2_ref, ow_ref, ob_ref, g_ref, bb_ref,
              f1w_ref, f1b_ref, f2w_ref, f2b_ref, y1_ref, y2_ref):
    a = (jnp.dot(a_ref[...], ow_ref[...], preferred_element_type=jnp.float32)
         + ob_ref[...])
    y1 = x1_ref[...].astype(jnp.float32) + a
    y1_ref[...] = y1.astype(jnp.bfloat16)
    h = _layer_norm(y1, g_ref[...], bb_ref[...])
    f = jax.nn.gelu(
        jnp.dot(h, f1w_ref[...], preferred_element_type=jnp.float32)
        + f1b_ref[...])
    f = (jnp.dot(f, f2w_ref[...], preferred_element_type=jnp.float32)
         + f2b_ref[...])
    y2_ref[...] = (x2_ref[...].astype(jnp.float32) + f).astype(jnp.bfloat16)


def _ffn(attn_out, x1, x2, ow, ob, g, b, f1w, f1b, f2w, f2b):
    nblk = 4
    rows = SEQ // nblk
    return pl.pallas_call(
        _ffn_body,
        grid=(nblk,),
        in_specs=[
            pl.BlockSpec((rows, EMBED), lambda i: (i, 0)),
            pl.BlockSpec((rows, EMBED), lambda i: (i, 0)),
            pl.BlockSpec((rows, EMBED), lambda i: (i, 0)),
            pl.BlockSpec((EMBED, EMBED), lambda i: (0, 0)),
            pl.BlockSpec((1, EMBED), lambda i: (0, 0)),
            pl.BlockSpec((1, EMBED), lambda i: (0, 0)),
            pl.BlockSpec((1, EMBED), lambda i: (0, 0)),
            pl.BlockSpec((EMBED, FF), lambda i: (0, 0)),
            pl.BlockSpec((1, FF), lambda i: (0, 0)),
            pl.BlockSpec((FF, EMBED), lambda i: (0, 0)),
            pl.BlockSpec((1, EMBED), lambda i: (0, 0)),
        ],
        out_specs=[
            pl.BlockSpec((rows, EMBED), lambda i: (i, 0)),
            pl.BlockSpec((rows, EMBED), lambda i: (i, 0)),
        ],
        out_shape=[
            jax.ShapeDtypeStruct((SEQ, EMBED), jnp.bfloat16),
            jax.ShapeDtypeStruct((SEQ, EMBED), jnp.bfloat16),
        ],
    )(attn_out, x1, x2, ow, ob, g, b, f1w, f1b, f2w, f2b)


def _final_body(x1_ref, x2_ref, keep_ref, pw_ref, pb_ref, cw_ref, cb_ref,
                o_ref):
    keep = keep_ref[...]
    hidden = ((x1_ref[...].astype(jnp.float32) + x2_ref[...].astype(jnp.float32)) * 0.5 * keep)
    pooled = jnp.sum(hidden, axis=0, keepdims=True) / jnp.sum(keep)
    pre = jnp.maximum(
        jnp.dot(pooled, pw_ref[...], preferred_element_type=jnp.float32)
        + pb_ref[...], 0.0)
    o_ref[...] = (
        jnp.dot(pre, cw_ref[...], preferred_element_type=jnp.float32)
        + cb_ref[...])


def _final(x1, x2, keep, pw, pb, cw, cb):
    return pl.pallas_call(
        _final_body,
        out_shape=jax.ShapeDtypeStruct((1, NUM_CLASSES), jnp.float32),
    )(x1, x2, keep, pw, pb, cw, cb)


def kernel(src, source_lengths, params):
    ids = src.reshape(-1)
    rows = _sc_gather(params["embed"], ids)
    pe_full = jnp.asarray(_pos_encoding_np(SEQ, EMBED))
    pe_zero = jnp.zeros((1, EMBED), jnp.float32)

    padf = (ids == 0).astype(jnp.float32)
    keep = (1.0 - padf).reshape(SEQ, 1)
    hm = jnp.asarray(
        np.kron(np.eye(HEADS, dtype=np.float32),
                np.ones((DH, DH), dtype=np.float32))).astype(jnp.bfloat16)

    x1 = rows
    x2 = rows
    pe = pe_full
    first = True
    for lp in params["layers"]:
        qs, kn, vaug, pdiag, xpe = _proj(
            x2, pe,
            lp["ln1_g"].reshape(1, EMBED), lp["ln1_b"].reshape(1, EMBED),
            lp["qk_w"], lp["qk_b"].reshape(1, EMBED),
            lp["v_w"], lp["v_b"].reshape(1, EMBED),
            hm, keep)
        attn_out = _attention(qs, kn, vaug, pdiag)
        if first:
            x1 = xpe
            x2 = xpe
            first = False
        x1, x2 = _ffn(
            attn_out, x1, x2,
            lp["o_w"], lp["o_b"].reshape(1, EMBED),
            lp["ln2_g"].reshape(1, EMBED), lp["ln2_b"].reshape(1, EMBED),
            lp["ff1_w"], lp["ff1_b"].reshape(1, FF),
            lp["ff2_w"], lp["ff2_b"].reshape(1, EMBED))
        pe = pe_zero

    return _final(
        x1, x2, keep,
        params["pre_w"], params["pre_b"].reshape(1, EMBED),
        params["cls_w"], params["cls_b"].reshape(1, NUM_CLASSES))
